# baseline (speedup 1.0000x reference)
import functools

import jax
import jax.numpy as jnp
from jax import lax
from jax.experimental import pallas as pl
from jax.experimental.pallas import tpu as pltpu
from jax.experimental.pallas import tpu_sc as plsc

NA = 10000
NI = 160000
D = 128
H = 4
F = 32
OUT = 256
ATTN = 128

NC = 2
NS = 16
NW = NC * NS

NI_PAD = 163840
EPT = NI_PAD // NW
ECH = 1024
NCH_A = EPT // ECH
KCH = ECH // 128
IROWS = NI_PAD // 128
RPT = IROWS // NW
NROWS = 10752
NJUNK = 512
ZPT = NROWS // NS
PTAB = 4 * NROWS

ROW_BLK = 1000

_mesh = plsc.VectorSubcoreMesh(core_axis_name="c", subcore_axis_name="s")
_sc_params = pltpu.CompilerParams(needs_layout_passes=False)
_sc_params_b = pltpu.CompilerParams(needs_layout_passes=False,
                                    use_tc_tiling_on_sc=False)


def _pass_a_body(tab_hbm, idx_hbm, s_out, ex_out,
                 pa_v, pb_v, i0b, i1b, i2b, exb0, exb1, exb2, exb3, zb,
                 s0, s1, s2, s3):
    cid = lax.axis_index("c")
    sid = lax.axis_index("s")
    wid = sid * NC + cid
    exbs = (exb0, exb1, exb2, exb3)
    ssh = (s0, s1, s2, s3)

    for t in range(ZPT // 16):
        zb[pl.ds(t * 16, 16)] = jnp.zeros((16,), jnp.float32)

    def mp_iter(mp, _):
        pltpu.sync_copy(tab_hbm.at[mp, 0], pa_v)
        pltpu.sync_copy(tab_hbm.at[mp, 1], pb_v)
        for s_sh in ssh:
            pltpu.sync_copy(zb, s_sh.at[pl.ds(sid * ZPT, ZPT)])
        plsc.subcore_barrier()

        def chunk(c, _):
            rb = wid * RPT + c * KCH
            pltpu.sync_copy(idx_hbm.at[mp, 0, pl.ds(rb, KCH)], i0b)
            pltpu.sync_copy(idx_hbm.at[mp, 1, pl.ds(rb, KCH)], i1b)
            pltpu.sync_copy(idx_hbm.at[mp, 2, pl.ds(rb, KCH)], i2b)

            def grp(g, _):
                k = g // 8
                m = g % 8
                off = m * 16
                b0 = i0b[k, pl.ds(off, 16)] * 4
                b1 = i1b[k, pl.ds(off, 16)] * 4
                b2 = i2b[k, pl.ds(off, 16)] * 4
                for h in range(H):
                    er = (plsc.load_gather(pa_v, [b0 + h])
                          + plsc.load_gather(pb_v, [b1 + h])
                          + plsc.load_gather(pa_v, [b2 + h])) * (1.0 / 3.0)
                    e = jnp.maximum(er, 0.01 * er)
                    exbs[h][k, pl.ds(off, 16)] = jnp.exp(e)
                return _

            lax.fori_loop(0, ECH // 16, grp, None)
            for h in range(H):
                pltpu.sync_copy(exbs[h], ex_out.at[mp, h, pl.ds(rb, KCH)])
            for k in range(KCH):
                for h in range(H):
                    pltpu.sync_copy(exbs[h].at[k], ssh[h].at[i0b.at[k]],
                                    add=True)
            return _

        lax.fori_loop(0, NCH_A, chunk, None)
        plsc.subcore_barrier()

        @pl.when(sid == 0)
        def _copy_out():
            for h in range(H):
                pltpu.sync_copy(ssh[h], s_out.at[mp, cid, h])

        plsc.subcore_barrier()
        return _

    lax.fori_loop(0, 2, mp_iter, None)


def _pass_a(tab, idxs):
    return pl.kernel(
        _pass_a_body,
        out_type=(
            jax.ShapeDtypeStruct((2, NC, H, NROWS), jnp.float32),
            jax.ShapeDtypeStruct((2, H, IROWS, 128), jnp.float32),
        ),
        mesh=_mesh,
        compiler_params=_sc_params,
        scratch_types=[
            pltpu.VMEM((PTAB,), jnp.float32),
            pltpu.VMEM((PTAB,), jnp.float32),
            pltpu.VMEM((KCH, 128), jnp.int32),
            pltpu.VMEM((KCH, 128), jnp.int32),
            pltpu.VMEM((KCH, 128), jnp.int32),
            pltpu.VMEM((KCH, 128), jnp.float32),
            pltpu.VMEM((KCH, 128), jnp.float32),
            pltpu.VMEM((KCH, 128), jnp.float32),
            pltpu.VMEM((KCH, 128), jnp.float32),
            pltpu.VMEM((ZPT,), jnp.float32),
            pltpu.VMEM_SHARED((NROWS,), jnp.float32),
            pltpu.VMEM_SHARED((NROWS,), jnp.float32),
            pltpu.VMEM_SHARED((NROWS,), jnp.float32),
            pltpu.VMEM_SHARED((NROWS,), jnp.float32),
        ],
    )(tab, idxs)


DH = D // 2


def _pass_b_body(hA_hbm, hrot_hbm, ex_hbm, s_hbm, idx_hbm, acc_out,
                 s0v, s1v, i0b, i1b, i2b, rA, rB,
                 wbuf0, wbuf1, exb0, exb1,
                 semA, semB, semS0, semS1, acc):
    cid = lax.axis_index("c")
    sid = lax.axis_index("s")
    wid = sid * NC + cid
    svs = (s0v, s1v)
    wbufs = (wbuf0, wbuf1)
    exbs = (exb0, exb1)

    def q_iter(q, _):
        mp = q // 2
        half = q % 2

        def zrow(t, _):
            for j in range(DH // 16):
                wbuf0[t, pl.ds(j * 16, 16)] = jnp.zeros((16,), jnp.float32)
                wbuf1[t, pl.ds(j * 16, 16)] = jnp.zeros((16,), jnp.float32)
            return _

        lax.fori_loop(0, 128, zrow, None)
        for t in range(5):
            pltpu.sync_copy(wbuf0, acc.at[pl.ds(sid * ZPT + t * 128, 128)])
        pltpu.sync_copy(wbuf0.at[pl.ds(0, 32)],
                        acc.at[pl.ds(sid * ZPT + 640, 32)])
        for hh in range(2):
            pltpu.sync_copy(s_hbm.at[mp, half * 2 + hh], svs[hh])
        pltpu.sync_copy(idx_hbm.at[mp, 0, pl.ds(wid * RPT, RPT)], i0b)
        pltpu.sync_copy(idx_hbm.at[mp, 1, pl.ds(wid * RPT, RPT)], i1b)
        pltpu.sync_copy(idx_hbm.at[mp, 2, pl.ds(wid * RPT, RPT)], i2b)
        for hh in range(2):
            pltpu.sync_copy(
                ex_hbm.at[mp, half * 2 + hh, pl.ds(wid * RPT, RPT)], exbs[hh])

        @plsc.parallel_loop(0, RPT * 8)
        def agrp(g):
            r = g // 8
            off = (g % 8) * 16
            i0v = i0b[r, pl.ds(off, 16)]
            for hh in range(2):
                sv = plsc.load_gather(svs[hh], [i0v])
                exbs[hh][r, pl.ds(off, 16)] = exbs[hh][r, pl.ds(off, 16)] / sv

        plsc.subcore_barrier()
        semSs = (semS0, semS1)
        for p in range(2):
            pltpu.async_copy(wbufs[p], acc.at[i0b.at[0]], semSs[p], add=True)

        def do_chunk(c, p):
            cpB = pltpu.async_copy(hrot_hbm.at[mp * 2 + half].at[i1b.at[c]],
                                   rB, semB)
            cpA = pltpu.async_copy(hA_hbm.at[half].at[i2b.at[c]], rA, semA)
            pltpu.make_async_copy(wbufs[p], acc.at[i0b.at[c]],
                                  semSs[p]).wait()
            cpB.wait()
            cpA.wait()
            cv = jnp.full((16,), c, jnp.int32)
            wb = wbufs[p]

            @plsc.parallel_loop(0, 128, unroll=4)
            def edge(e):
                ev = jnp.full((16,), e, jnp.int32)
                sp = [plsc.load_gather(exbs[hh], [cv, ev]) for hh in range(2)]
                for j in range(DH // 16):
                    w = (rB[e, pl.ds(j * 16, 16)]
                         + rA[e, pl.ds(j * 16, 16)]) * sp[j // 2]
                    wb[e, pl.ds(j * 16, 16)] = w

            pltpu.async_copy(wb, acc.at[i0b.at[c]], semSs[p], add=True)

        def chunk_pair(t, _):
            for p in range(2):
                do_chunk(2 * t + p, p)
            return _

        lax.fori_loop(0, RPT // 2, chunk_pair, None)
        for p in range(2):
            pltpu.make_async_copy(wbufs[p], acc.at[i0b.at[0]], semSs[p]).wait()
        plsc.subcore_barrier()
        pltpu.sync_copy(acc.at[pl.ds(sid * ZPT, ZPT)],
                        acc_out.at[mp, half, cid, pl.ds(sid * ZPT, ZPT)])
        plsc.subcore_barrier()
        return _

    lax.fori_loop(0, 4, q_iter, None)


def _pass_b(hAh, hroth, ex, s_tot, idxs):
    return pl.kernel(
        _pass_b_body,
        out_type=jax.ShapeDtypeStruct((2, 2, NC, NROWS, DH), jnp.float32),
        mesh=_mesh,
        compiler_params=_sc_params_b,
        scratch_types=[
            pltpu.VMEM((NROWS,), jnp.float32),
            pltpu.VMEM((NROWS,), jnp.float32),
            pltpu.VMEM((RPT, 128), jnp.int32),
            pltpu.VMEM((RPT, 128), jnp.int32),
            pltpu.VMEM((RPT, 128), jnp.int32),
            pltpu.VMEM((128, DH), jnp.float32),
            pltpu.VMEM((128, DH), jnp.float32),
            pltpu.VMEM((128, DH), jnp.float32),
            pltpu.VMEM((128, DH), jnp.float32),
            pltpu.VMEM((RPT, 128), jnp.float32),
            pltpu.VMEM((RPT, 128), jnp.float32),
            pltpu.SemaphoreType.DMA,
            pltpu.SemaphoreType.DMA,
            pltpu.SemaphoreType.DMA,
            pltpu.SemaphoreType.DMA,
            pltpu.VMEM_SHARED((NROWS, DH), jnp.float32),
        ],
    )(hAh, hroth, ex, s_tot, idxs)


def _tail_reduce_kernel(fa1_ref, fa2_ref, wl_ref, bl_ref, out_ref):
    i = pl.program_id(0)
    t1 = jnp.tanh(jnp.dot(fa1_ref[...], wl_ref[...],
                          preferred_element_type=jnp.float32) + bl_ref[...])
    t2 = jnp.tanh(jnp.dot(fa2_ref[...], wl_ref[...],
                          preferred_element_type=jnp.float32) + bl_ref[...])
    part = jnp.stack([jnp.sum(t1, axis=0), jnp.sum(t2, axis=0)], axis=0)

    @pl.when(i == 0)
    def _init():
        out_ref[...] = jnp.zeros_like(out_ref)

    out_ref[...] += part


def _tail_out_kernel(beta_ref, fa1_ref, fa2_ref, hb_ref, hc_ref, wo_ref,
                     bo_ref, featA_ref, outA_ref, outB_ref, outC_ref):
    b0 = beta_ref[0]
    b1 = beta_ref[1]
    featA = b0 * fa1_ref[...] + b1 * fa2_ref[...]
    featA_ref[...] = featA
    wo = wo_ref[...]
    bo = bo_ref[...]
    outA_ref[...] = jnp.dot(featA, wo, preferred_element_type=jnp.float32) + bo
    outB_ref[...] = jnp.dot(hb_ref[...], wo, preferred_element_type=jnp.float32) + bo
    outC_ref[...] = jnp.dot(hc_ref[...], wo, preferred_element_type=jnp.float32) + bo


def _tail(fa1, fa2, hB, hC, Wl, bl, v, W_out, b_out):
    grid = NA // ROW_BLK
    tsum = pl.pallas_call(
        _tail_reduce_kernel,
        grid=(grid,),
        in_specs=[
            pl.BlockSpec((ROW_BLK, D), lambda i: (i, 0)),
            pl.BlockSpec((ROW_BLK, D), lambda i: (i, 0)),
            pl.BlockSpec((D, ATTN), lambda i: (0, 0)),
            pl.BlockSpec((1, ATTN), lambda i: (0, 0)),
        ],
        out_specs=pl.BlockSpec((2, ATTN), lambda i: (0, 0)),
        out_shape=jax.ShapeDtypeStruct((2, ATTN), jnp.float32),
    )(fa1, fa2, Wl, bl.reshape(1, ATTN))
    scores = (tsum / NA) @ v
    beta = jax.nn.softmax(scores)
    featA, outA, outB, outC = pl.pallas_call(
        _tail_out_kernel,
        grid=(grid,),
        in_specs=[
            pl.BlockSpec(memory_space=pltpu.SMEM),
            pl.BlockSpec((ROW_BLK, D), lambda i: (i, 0)),
            pl.BlockSpec((ROW_BLK, D), lambda i: (i, 0)),
            pl.BlockSpec((ROW_BLK, D), lambda i: (i, 0)),
            pl.BlockSpec((ROW_BLK, D), lambda i: (i, 0)),
            pl.BlockSpec((D, OUT), lambda i: (0, 0)),
            pl.BlockSpec((1, OUT), lambda i: (0, 0)),
        ],
        out_specs=[
            pl.BlockSpec((ROW_BLK, D), lambda i: (i, 0)),
            pl.BlockSpec((ROW_BLK, OUT), lambda i: (i, 0)),
            pl.BlockSpec((ROW_BLK, OUT), lambda i: (i, 0)),
            pl.BlockSpec((ROW_BLK, OUT), lambda i: (i, 0)),
        ],
        out_shape=[
            jax.ShapeDtypeStruct((NA, D), jnp.float32),
            jax.ShapeDtypeStruct((NA, OUT), jnp.float32),
            jax.ShapeDtypeStruct((NA, OUT), jnp.float32),
            jax.ShapeDtypeStruct((NA, OUT), jnp.float32),
        ],
    )(beta, fa1, fa2, hB, hC, W_out, b_out.reshape(1, OUT))
    return featA, outA, outB, outC


def _rotate(h, r):
    hc = h.reshape(-1, D // 2, 2)
    hr, hi = hc[:, :, 0], hc[:, :, 1]
    rr, ri = r[:, 0], r[:, 1]
    return jnp.stack([hr * rr - hi * ri, hr * ri + hi * rr], axis=2).reshape(-1, D)


def _pad_cols(idx):
    npad = NI_PAD - NI
    j = jnp.arange(npad, dtype=jnp.int32)
    i0 = jnp.concatenate([idx[:, 0].astype(jnp.int32), NA + (j % NJUNK)])
    i1 = jnp.concatenate([idx[:, 1].astype(jnp.int32), j % NA])
    i2 = jnp.concatenate([idx[:, 2].astype(jnp.int32), j % NA])
    return (i0.reshape(IROWS, 128), i1.reshape(IROWS, 128),
            i2.reshape(IROWS, 128))


def _ptab(p):
    return jnp.concatenate(
        [p.reshape(NA * H), jnp.zeros((PTAB - NA * H,), jnp.float32)])


def kernel(hA, hB, hC, idx_ABA, idx_ACA, attn_r_ABA, attn_r_ACA,
           rAB, rBA, rAC, rCA, Wl, bl, v, W_out, b_out):
    hBrot = _rotate(hB, rAB)
    hCrot = _rotate(hC, rAC)
    hA4 = hA.reshape(NA, H, F)
    pa1 = jnp.einsum("nhf,hf->nh", hA4, attn_r_ABA[0])
    pb1 = jnp.einsum("nhf,hf->nh", hBrot.reshape(NA, H, F), attn_r_ABA[0])
    pa2 = jnp.einsum("nhf,hf->nh", hA4, attn_r_ACA[0])
    pc2 = jnp.einsum("nhf,hf->nh", hCrot.reshape(NA, H, F), attn_r_ACA[0])

    tab = jnp.stack([jnp.stack([_ptab(pa1), _ptab(pb1)]),
                     jnp.stack([_ptab(pa2), _ptab(pc2)])])
    idxs = jnp.stack([jnp.stack(_pad_cols(idx_ABA)),
                      jnp.stack(_pad_cols(idx_ACA))])

    s_pair, ex = _pass_a(tab, idxs)
    s_tot = s_pair[:, 0] + s_pair[:, 1]
    hAh = jnp.stack([hA[:, :DH], hA[:, DH:]])
    hroth = jnp.stack([hBrot[:, :DH], hBrot[:, DH:],
                       hCrot[:, :DH], hCrot[:, DH:]])
    acc = _pass_b(hAh, hroth, ex, s_tot, idxs)
    accs = acc[:, :, 0] + acc[:, :, 1]
    h_raw = jnp.concatenate([accs[:, 0], accs[:, 1]], axis=-1)[:, :NA]
    has = (s_tot[:, 0, :NA] > 0.0).astype(jnp.float32)
    fa = jax.nn.elu((h_raw + hA[None] * has[:, :, None]) * (1.0 / 3.0))

    featA, outA, outB, outC = _tail(fa[0], fa[1], hB, hC, Wl, bl, v,
                                    W_out, b_out)
    return (outA, outB, outC, featA, hB, hC)


# speedup vs baseline: 1.0108x; 1.0108x over previous
import functools

import jax
import jax.numpy as jnp
from jax import lax
from jax.experimental import pallas as pl
from jax.experimental.pallas import tpu as pltpu
from jax.experimental.pallas import tpu_sc as plsc

NA = 10000
NI = 160000
D = 128
H = 4
F = 32
OUT = 256
ATTN = 128

NC = 2
NS = 16
NW = NC * NS

NI_PAD = 163840
EPT = NI_PAD // NW
ECH = 1024
NCH_A = EPT // ECH
KCH = ECH // 128
IROWS = NI_PAD // 128
RPT = IROWS // NW
NROWS = 10752
NJUNK = 512
ZPT = NROWS // NS
PTAB = 4 * NROWS

ROW_BLK = 1000

_mesh = plsc.VectorSubcoreMesh(core_axis_name="c", subcore_axis_name="s")
_sc_params = pltpu.CompilerParams(needs_layout_passes=False)
_sc_params_b = pltpu.CompilerParams(needs_layout_passes=False,
                                    use_tc_tiling_on_sc=False)


def _pass_a_body(tab_hbm, idx_hbm, s_out, ex_out,
                 pa_v, pb_v, i0b, i1b, i2b, exb0, exb1, exb2, exb3, zb,
                 s0, s1, s2, s3):
    cid = lax.axis_index("c")
    sid = lax.axis_index("s")
    wid = sid * NC + cid
    exbs = (exb0, exb1, exb2, exb3)
    ssh = (s0, s1, s2, s3)

    for t in range(ZPT // 16):
        zb[pl.ds(t * 16, 16)] = jnp.zeros((16,), jnp.float32)

    def mp_iter(mp, _):
        pltpu.sync_copy(tab_hbm.at[mp, 0], pa_v)
        pltpu.sync_copy(tab_hbm.at[mp, 1], pb_v)
        for s_sh in ssh:
            pltpu.sync_copy(zb, s_sh.at[pl.ds(sid * ZPT, ZPT)])
        plsc.subcore_barrier()

        def chunk(c, _):
            rb = wid * RPT + c * KCH
            pltpu.sync_copy(idx_hbm.at[mp, 0, pl.ds(rb, KCH)], i0b)
            pltpu.sync_copy(idx_hbm.at[mp, 1, pl.ds(rb, KCH)], i1b)
            pltpu.sync_copy(idx_hbm.at[mp, 2, pl.ds(rb, KCH)], i2b)

            def grp(g, _):
                k = g // 8
                m = g % 8
                off = m * 16
                b0 = i0b[k, pl.ds(off, 16)] * 4
                b1 = i1b[k, pl.ds(off, 16)] * 4
                b2 = i2b[k, pl.ds(off, 16)] * 4
                for h in range(H):
                    er = (plsc.load_gather(pa_v, [b0 + h])
                          + plsc.load_gather(pb_v, [b1 + h])
                          + plsc.load_gather(pa_v, [b2 + h])) * (1.0 / 3.0)
                    e = jnp.maximum(er, 0.01 * er)
                    exbs[h][k, pl.ds(off, 16)] = jnp.exp(e)
                return _

            lax.fori_loop(0, ECH // 16, grp, None)
            for h in range(H):
                pltpu.sync_copy(exbs[h], ex_out.at[mp, h, pl.ds(rb, KCH)])
            for k in range(KCH):
                for h in range(H):
                    pltpu.sync_copy(exbs[h].at[k], ssh[h].at[i0b.at[k]],
                                    add=True)
            return _

        lax.fori_loop(0, NCH_A, chunk, None)
        plsc.subcore_barrier()

        @pl.when(sid == 0)
        def _copy_out():
            for h in range(H):
                pltpu.sync_copy(ssh[h], s_out.at[mp, cid, h])

        plsc.subcore_barrier()
        return _

    lax.fori_loop(0, 2, mp_iter, None)


def _pass_a(tab, idxs):
    return pl.kernel(
        _pass_a_body,
        out_type=(
            jax.ShapeDtypeStruct((2, NC, H, NROWS), jnp.float32),
            jax.ShapeDtypeStruct((2, H, IROWS, 128), jnp.float32),
        ),
        mesh=_mesh,
        compiler_params=_sc_params,
        scratch_types=[
            pltpu.VMEM((PTAB,), jnp.float32),
            pltpu.VMEM((PTAB,), jnp.float32),
            pltpu.VMEM((KCH, 128), jnp.int32),
            pltpu.VMEM((KCH, 128), jnp.int32),
            pltpu.VMEM((KCH, 128), jnp.int32),
            pltpu.VMEM((KCH, 128), jnp.float32),
            pltpu.VMEM((KCH, 128), jnp.float32),
            pltpu.VMEM((KCH, 128), jnp.float32),
            pltpu.VMEM((KCH, 128), jnp.float32),
            pltpu.VMEM((ZPT,), jnp.float32),
            pltpu.VMEM_SHARED((NROWS,), jnp.float32),
            pltpu.VMEM_SHARED((NROWS,), jnp.float32),
            pltpu.VMEM_SHARED((NROWS,), jnp.float32),
            pltpu.VMEM_SHARED((NROWS,), jnp.float32),
        ],
    )(tab, idxs)


DH = D // 2


def _pass_b_body(hA_hbm, hrot_hbm, ex_hbm, s_hbm, idx_hbm, acc_out,
                 s0v, s1v, i0b, i1b, i2b, rA, rB,
                 wbuf0, wbuf1, exb0, exb1,
                 semA, semB, semS0, semS1, acc):
    cid = lax.axis_index("c")
    sid = lax.axis_index("s")
    wid = sid * NC + cid
    svs = (s0v, s1v)
    wbufs = (wbuf0, wbuf1)
    exbs = (exb0, exb1)

    def q_iter(q, _):
        mp = q // 2
        half = q % 2

        def zrow(t, _):
            for j in range(DH // 16):
                wbuf0[t, pl.ds(j * 16, 16)] = jnp.zeros((16,), jnp.float32)
                wbuf1[t, pl.ds(j * 16, 16)] = jnp.zeros((16,), jnp.float32)
            return _

        lax.fori_loop(0, 128, zrow, None)
        for t in range(5):
            pltpu.sync_copy(wbuf0, acc.at[pl.ds(sid * ZPT + t * 128, 128)])
        pltpu.sync_copy(wbuf0.at[pl.ds(0, 32)],
                        acc.at[pl.ds(sid * ZPT + 640, 32)])
        for hh in range(2):
            pltpu.sync_copy(s_hbm.at[mp, half * 2 + hh], svs[hh])
        pltpu.sync_copy(idx_hbm.at[mp, 0, pl.ds(wid * RPT, RPT)], i0b)
        pltpu.sync_copy(idx_hbm.at[mp, 1, pl.ds(wid * RPT, RPT)], i1b)
        pltpu.sync_copy(idx_hbm.at[mp, 2, pl.ds(wid * RPT, RPT)], i2b)
        for hh in range(2):
            pltpu.sync_copy(
                ex_hbm.at[mp, half * 2 + hh, pl.ds(wid * RPT, RPT)], exbs[hh])
        plsc.subcore_barrier()
        semSs = (semS0, semS1)
        for p in range(2):
            pltpu.async_copy(wbufs[p], acc.at[i0b.at[0]], semSs[p], add=True)

        def do_chunk(c, p):
            cpB = pltpu.async_copy(hrot_hbm.at[mp * 2 + half].at[i1b.at[c]],
                                   rB, semB)
            cpA = pltpu.async_copy(hA_hbm.at[half].at[i2b.at[c]], rA, semA)

            @plsc.parallel_loop(0, 8)
            def agrp(m):
                off = m * 16
                i0v = i0b[c, pl.ds(off, 16)]
                for hh in range(2):
                    sv = plsc.load_gather(svs[hh], [i0v])
                    exbs[hh][c, pl.ds(off, 16)] = (
                        exbs[hh][c, pl.ds(off, 16)] / sv)

            pltpu.make_async_copy(wbufs[p], acc.at[i0b.at[c]],
                                  semSs[p]).wait()
            cpB.wait()
            cpA.wait()
            cv = jnp.full((16,), c, jnp.int32)
            wb = wbufs[p]

            @plsc.parallel_loop(0, 128, unroll=4)
            def edge(e):
                ev = jnp.full((16,), e, jnp.int32)
                sp = [plsc.load_gather(exbs[hh], [cv, ev]) for hh in range(2)]
                for j in range(DH // 16):
                    w = (rB[e, pl.ds(j * 16, 16)]
                         + rA[e, pl.ds(j * 16, 16)]) * sp[j // 2]
                    wb[e, pl.ds(j * 16, 16)] = w

            pltpu.async_copy(wb, acc.at[i0b.at[c]], semSs[p], add=True)

        def chunk_pair(t, _):
            for p in range(2):
                do_chunk(2 * t + p, p)
            return _

        lax.fori_loop(0, RPT // 2, chunk_pair, None)
        for p in range(2):
            pltpu.make_async_copy(wbufs[p], acc.at[i0b.at[0]], semSs[p]).wait()
        plsc.subcore_barrier()
        pltpu.sync_copy(acc.at[pl.ds(sid * ZPT, ZPT)],
                        acc_out.at[mp, half, cid, pl.ds(sid * ZPT, ZPT)])
        plsc.subcore_barrier()
        return _

    lax.fori_loop(0, 4, q_iter, None)


def _pass_b(hAh, hroth, ex, s_tot, idxs):
    return pl.kernel(
        _pass_b_body,
        out_type=jax.ShapeDtypeStruct((2, 2, NC, NROWS, DH), jnp.float32),
        mesh=_mesh,
        compiler_params=_sc_params_b,
        scratch_types=[
            pltpu.VMEM((NROWS,), jnp.float32),
            pltpu.VMEM((NROWS,), jnp.float32),
            pltpu.VMEM((RPT, 128), jnp.int32),
            pltpu.VMEM((RPT, 128), jnp.int32),
            pltpu.VMEM((RPT, 128), jnp.int32),
            pltpu.VMEM((128, DH), jnp.float32),
            pltpu.VMEM((128, DH), jnp.float32),
            pltpu.VMEM((128, DH), jnp.float32),
            pltpu.VMEM((128, DH), jnp.float32),
            pltpu.VMEM((RPT, 128), jnp.float32),
            pltpu.VMEM((RPT, 128), jnp.float32),
            pltpu.SemaphoreType.DMA,
            pltpu.SemaphoreType.DMA,
            pltpu.SemaphoreType.DMA,
            pltpu.SemaphoreType.DMA,
            pltpu.VMEM_SHARED((NROWS, DH), jnp.float32),
        ],
    )(hAh, hroth, ex, s_tot, idxs)


def _tail_reduce_kernel(fa1_ref, fa2_ref, wl_ref, bl_ref, out_ref):
    i = pl.program_id(0)
    t1 = jnp.tanh(jnp.dot(fa1_ref[...], wl_ref[...],
                          preferred_element_type=jnp.float32) + bl_ref[...])
    t2 = jnp.tanh(jnp.dot(fa2_ref[...], wl_ref[...],
                          preferred_element_type=jnp.float32) + bl_ref[...])
    part = jnp.stack([jnp.sum(t1, axis=0), jnp.sum(t2, axis=0)], axis=0)

    @pl.when(i == 0)
    def _init():
        out_ref[...] = jnp.zeros_like(out_ref)

    out_ref[...] += part


def _tail_out_kernel(beta_ref, fa1_ref, fa2_ref, hb_ref, hc_ref, wo_ref,
                     bo_ref, featA_ref, outA_ref, outB_ref, outC_ref):
    b0 = beta_ref[0]
    b1 = beta_ref[1]
    featA = b0 * fa1_ref[...] + b1 * fa2_ref[...]
    featA_ref[...] = featA
    wo = wo_ref[...]
    bo = bo_ref[...]
    outA_ref[...] = jnp.dot(featA, wo, preferred_element_type=jnp.float32) + bo
    outB_ref[...] = jnp.dot(hb_ref[...], wo, preferred_element_type=jnp.float32) + bo
    outC_ref[...] = jnp.dot(hc_ref[...], wo, preferred_element_type=jnp.float32) + bo


def _tail(fa1, fa2, hB, hC, Wl, bl, v, W_out, b_out):
    grid = NA // ROW_BLK
    tsum = pl.pallas_call(
        _tail_reduce_kernel,
        grid=(grid,),
        in_specs=[
            pl.BlockSpec((ROW_BLK, D), lambda i: (i, 0)),
            pl.BlockSpec((ROW_BLK, D), lambda i: (i, 0)),
            pl.BlockSpec((D, ATTN), lambda i: (0, 0)),
            pl.BlockSpec((1, ATTN), lambda i: (0, 0)),
        ],
        out_specs=pl.BlockSpec((2, ATTN), lambda i: (0, 0)),
        out_shape=jax.ShapeDtypeStruct((2, ATTN), jnp.float32),
    )(fa1, fa2, Wl, bl.reshape(1, ATTN))
    scores = (tsum / NA) @ v
    beta = jax.nn.softmax(scores)
    featA, outA, outB, outC = pl.pallas_call(
        _tail_out_kernel,
        grid=(grid,),
        in_specs=[
            pl.BlockSpec(memory_space=pltpu.SMEM),
            pl.BlockSpec((ROW_BLK, D), lambda i: (i, 0)),
            pl.BlockSpec((ROW_BLK, D), lambda i: (i, 0)),
            pl.BlockSpec((ROW_BLK, D), lambda i: (i, 0)),
            pl.BlockSpec((ROW_BLK, D), lambda i: (i, 0)),
            pl.BlockSpec((D, OUT), lambda i: (0, 0)),
            pl.BlockSpec((1, OUT), lambda i: (0, 0)),
        ],
        out_specs=[
            pl.BlockSpec((ROW_BLK, D), lambda i: (i, 0)),
            pl.BlockSpec((ROW_BLK, OUT), lambda i: (i, 0)),
            pl.BlockSpec((ROW_BLK, OUT), lambda i: (i, 0)),
            pl.BlockSpec((ROW_BLK, OUT), lambda i: (i, 0)),
        ],
        out_shape=[
            jax.ShapeDtypeStruct((NA, D), jnp.float32),
            jax.ShapeDtypeStruct((NA, OUT), jnp.float32),
            jax.ShapeDtypeStruct((NA, OUT), jnp.float32),
            jax.ShapeDtypeStruct((NA, OUT), jnp.float32),
        ],
    )(beta, fa1, fa2, hB, hC, W_out, b_out.reshape(1, OUT))
    return featA, outA, outB, outC


def _rotate(h, r):
    hc = h.reshape(-1, D // 2, 2)
    hr, hi = hc[:, :, 0], hc[:, :, 1]
    rr, ri = r[:, 0], r[:, 1]
    return jnp.stack([hr * rr - hi * ri, hr * ri + hi * rr], axis=2).reshape(-1, D)


def _pad_cols(idx):
    npad = NI_PAD - NI
    j = jnp.arange(npad, dtype=jnp.int32)
    i0 = jnp.concatenate([idx[:, 0].astype(jnp.int32), NA + (j % NJUNK)])
    i1 = jnp.concatenate([idx[:, 1].astype(jnp.int32), j % NA])
    i2 = jnp.concatenate([idx[:, 2].astype(jnp.int32), j % NA])
    return (i0.reshape(IROWS, 128), i1.reshape(IROWS, 128),
            i2.reshape(IROWS, 128))


def _ptab(p):
    return jnp.concatenate(
        [p.reshape(NA * H), jnp.zeros((PTAB - NA * H,), jnp.float32)])


def kernel(hA, hB, hC, idx_ABA, idx_ACA, attn_r_ABA, attn_r_ACA,
           rAB, rBA, rAC, rCA, Wl, bl, v, W_out, b_out):
    hBrot = _rotate(hB, rAB)
    hCrot = _rotate(hC, rAC)
    hA4 = hA.reshape(NA, H, F)
    pa1 = jnp.einsum("nhf,hf->nh", hA4, attn_r_ABA[0])
    pb1 = jnp.einsum("nhf,hf->nh", hBrot.reshape(NA, H, F), attn_r_ABA[0])
    pa2 = jnp.einsum("nhf,hf->nh", hA4, attn_r_ACA[0])
    pc2 = jnp.einsum("nhf,hf->nh", hCrot.reshape(NA, H, F), attn_r_ACA[0])

    tab = jnp.stack([jnp.stack([_ptab(pa1), _ptab(pb1)]),
                     jnp.stack([_ptab(pa2), _ptab(pc2)])])
    idxs = jnp.stack([jnp.stack(_pad_cols(idx_ABA)),
                      jnp.stack(_pad_cols(idx_ACA))])

    s_pair, ex = _pass_a(tab, idxs)
    s_tot = s_pair[:, 0] + s_pair[:, 1]
    hAh = jnp.stack([hA[:, :DH], hA[:, DH:]])
    hroth = jnp.stack([hBrot[:, :DH], hBrot[:, DH:],
                       hCrot[:, :DH], hCrot[:, DH:]])
    acc = _pass_b(hAh, hroth, ex, s_tot, idxs)
    accs = acc[:, :, 0] + acc[:, :, 1]
    h_raw = jnp.concatenate([accs[:, 0], accs[:, 1]], axis=-1)[:, :NA]
    has = (s_tot[:, 0, :NA] > 0.0).astype(jnp.float32)
    fa = jax.nn.elu((h_raw + hA[None] * has[:, :, None]) * (1.0 / 3.0))

    featA, outA, outB, outC = _tail(fa[0], fa[1], hB, hC, Wl, bl, v,
                                    W_out, b_out)
    return (outA, outB, outC, featA, hB, hC)


# speedup vs baseline: 1.0530x; 1.0418x over previous
import functools

import jax
import jax.numpy as jnp
from jax import lax
from jax.experimental import pallas as pl
from jax.experimental.pallas import tpu as pltpu
from jax.experimental.pallas import tpu_sc as plsc

NA = 10000
NI = 160000
D = 128
H = 4
F = 32
OUT = 256
ATTN = 128

NC = 2
NS = 16
NW = NC * NS

NI_PAD = 163840
EPT = NI_PAD // NW
ECH = 1024
NCH_A = EPT // ECH
KCH = ECH // 128
IROWS = NI_PAD // 128
RPT = IROWS // NW
NROWS = 10752
NJUNK = 512
ZPT = NROWS // NS
PTAB = 4 * NROWS

ROW_BLK = 1000

_mesh = plsc.VectorSubcoreMesh(core_axis_name="c", subcore_axis_name="s")
_sc_params = pltpu.CompilerParams(needs_layout_passes=False)
_sc_params_b = pltpu.CompilerParams(needs_layout_passes=False,
                                    use_tc_tiling_on_sc=False)


def _pass_a_body(tab_hbm, idx_hbm, s_out, ex_out,
                 pa_v, pb_v, i0b, i1b, i2b, exb0, exb1, exb2, exb3, zb,
                 s0, s1, s2, s3):
    cid = lax.axis_index("c")
    sid = lax.axis_index("s")
    wid = sid * NC + cid
    exbs = (exb0, exb1, exb2, exb3)
    ssh = (s0, s1, s2, s3)

    for t in range(ZPT // 16):
        zb[pl.ds(t * 16, 16)] = jnp.zeros((16,), jnp.float32)

    def mp_iter(mp, _):
        pltpu.sync_copy(tab_hbm.at[mp, 0], pa_v)
        pltpu.sync_copy(tab_hbm.at[mp, 1], pb_v)
        for s_sh in ssh:
            pltpu.sync_copy(zb, s_sh.at[pl.ds(sid * ZPT, ZPT)])
        plsc.subcore_barrier()

        def chunk(c, _):
            rb = wid * RPT + c * KCH
            pltpu.sync_copy(idx_hbm.at[mp, 0, pl.ds(rb, KCH)], i0b)
            pltpu.sync_copy(idx_hbm.at[mp, 1, pl.ds(rb, KCH)], i1b)
            pltpu.sync_copy(idx_hbm.at[mp, 2, pl.ds(rb, KCH)], i2b)

            def grp(g, _):
                k = g // 8
                m = g % 8
                off = m * 16
                b0 = i0b[k, pl.ds(off, 16)] * 4
                b1 = i1b[k, pl.ds(off, 16)] * 4
                b2 = i2b[k, pl.ds(off, 16)] * 4
                for h in range(H):
                    er = (plsc.load_gather(pa_v, [b0 + h])
                          + plsc.load_gather(pb_v, [b1 + h])
                          + plsc.load_gather(pa_v, [b2 + h])) * (1.0 / 3.0)
                    e = jnp.maximum(er, 0.01 * er)
                    exbs[h][k, pl.ds(off, 16)] = jnp.exp(e)
                return _

            lax.fori_loop(0, ECH // 16, grp, None)
            for h in range(H):
                pltpu.sync_copy(exbs[h], ex_out.at[mp, h, pl.ds(rb, KCH)])
            for k in range(KCH):
                for h in range(H):
                    pltpu.sync_copy(exbs[h].at[k], ssh[h].at[i0b.at[k]],
                                    add=True)
            return _

        lax.fori_loop(0, NCH_A, chunk, None)
        plsc.subcore_barrier()

        @pl.when(sid == 0)
        def _copy_out():
            for h in range(H):
                pltpu.sync_copy(ssh[h], s_out.at[mp, cid, h])

        plsc.subcore_barrier()
        return _

    lax.fori_loop(0, 2, mp_iter, None)


def _pass_a(tab, idxs):
    return pl.kernel(
        _pass_a_body,
        out_type=(
            jax.ShapeDtypeStruct((2, NC, H, NROWS), jnp.float32),
            jax.ShapeDtypeStruct((2, H, IROWS, 128), jnp.float32),
        ),
        mesh=_mesh,
        compiler_params=_sc_params,
        scratch_types=[
            pltpu.VMEM((PTAB,), jnp.float32),
            pltpu.VMEM((PTAB,), jnp.float32),
            pltpu.VMEM((KCH, 128), jnp.int32),
            pltpu.VMEM((KCH, 128), jnp.int32),
            pltpu.VMEM((KCH, 128), jnp.int32),
            pltpu.VMEM((KCH, 128), jnp.float32),
            pltpu.VMEM((KCH, 128), jnp.float32),
            pltpu.VMEM((KCH, 128), jnp.float32),
            pltpu.VMEM((KCH, 128), jnp.float32),
            pltpu.VMEM((ZPT,), jnp.float32),
            pltpu.VMEM_SHARED((NROWS,), jnp.float32),
            pltpu.VMEM_SHARED((NROWS,), jnp.float32),
            pltpu.VMEM_SHARED((NROWS,), jnp.float32),
            pltpu.VMEM_SHARED((NROWS,), jnp.float32),
        ],
    )(tab, idxs)


DH = D // 2


def _pass_b_body(hA_hbm, hrot_hbm, ex_hbm, s_hbm, idx_hbm, acc_out,
                 s0v, s1v, i0b, i1b, i2b, rA, rB,
                 wbuf0, wbuf1, exb0, exb1,
                 semA, semB, semS0, semS1, acc):
    cid = lax.axis_index("c")
    sid = lax.axis_index("s")
    wid = sid * NC + cid
    svs = (s0v, s1v)
    wbufs = (wbuf0, wbuf1)
    exbs = (exb0, exb1)

    def q_iter(q, _):
        mp = q // 2
        half = q % 2

        def zrow(t, _):
            for j in range(DH // 16):
                wbuf0[t, pl.ds(j * 16, 16)] = jnp.zeros((16,), jnp.float32)
                wbuf1[t, pl.ds(j * 16, 16)] = jnp.zeros((16,), jnp.float32)
            return _

        lax.fori_loop(0, 128, zrow, None)
        for t in range(5):
            pltpu.sync_copy(wbuf0, acc.at[pl.ds(sid * ZPT + t * 128, 128)])
        pltpu.sync_copy(wbuf0.at[pl.ds(0, 32)],
                        acc.at[pl.ds(sid * ZPT + 640, 32)])
        for hh in range(2):
            pltpu.sync_copy(s_hbm.at[mp, half * 2 + hh], svs[hh])
        pltpu.sync_copy(idx_hbm.at[mp, 0, pl.ds(wid * RPT, RPT)], i0b)
        pltpu.sync_copy(idx_hbm.at[mp, 1, pl.ds(wid * RPT, RPT)], i1b)
        pltpu.sync_copy(idx_hbm.at[mp, 2, pl.ds(wid * RPT, RPT)], i2b)
        for hh in range(2):
            pltpu.sync_copy(
                ex_hbm.at[mp, half * 2 + hh, pl.ds(wid * RPT, RPT)], exbs[hh])
        plsc.subcore_barrier()
        semSs = (semS0, semS1)
        for p in range(2):
            pltpu.async_copy(wbufs[p], acc.at[i0b.at[0]], semSs[p], add=True)

        def do_chunk(c, p):
            cpB = pltpu.async_copy(hrot_hbm.at[mp * 2 + half].at[i1b.at[c]],
                                   rB, semB)
            cpA = pltpu.async_copy(hA_hbm.at[half].at[i2b.at[c]], rA, semA)

            @plsc.parallel_loop(0, 8)
            def agrp(m):
                off = m * 16
                i0v = i0b[c, pl.ds(off, 16)]
                for hh in range(2):
                    sv = plsc.load_gather(svs[hh], [i0v])
                    exbs[hh][c, pl.ds(off, 16)] = (
                        exbs[hh][c, pl.ds(off, 16)] / sv)

            pltpu.make_async_copy(wbufs[p], acc.at[i0b.at[c]],
                                  semSs[p]).wait()
            cpB.wait()
            cpA.wait()
            cv = jnp.full((16,), c, jnp.int32)
            wb = wbufs[p]

            @plsc.parallel_loop(0, 128, unroll=4)
            def edge(e):
                ev = jnp.full((16,), e, jnp.int32)
                sp = [plsc.load_gather(exbs[hh], [cv, ev]) for hh in range(2)]
                for j in range(DH // 16):
                    w = (rB[e, pl.ds(j * 16, 16)]
                         + rA[e, pl.ds(j * 16, 16)]) * sp[j // 2]
                    wb[e, pl.ds(j * 16, 16)] = w

            pltpu.async_copy(wb, acc.at[i0b.at[c]], semSs[p], add=True)

        def chunk_pair(t, _):
            for p in range(2):
                do_chunk(2 * t + p, p)
            return _

        lax.fori_loop(0, RPT // 2, chunk_pair, None)
        for p in range(2):
            pltpu.make_async_copy(wbufs[p], acc.at[i0b.at[0]], semSs[p]).wait()
        plsc.subcore_barrier()
        pltpu.sync_copy(acc.at[pl.ds(sid * ZPT, ZPT)],
                        acc_out.at[mp, half, cid, pl.ds(sid * ZPT, ZPT)])
        plsc.subcore_barrier()
        return _

    lax.fori_loop(0, 4, q_iter, None)


def _pass_b(hAh, hroth, ex, s_tot, idxs):
    return pl.kernel(
        _pass_b_body,
        out_type=jax.ShapeDtypeStruct((2, 2, NC, NROWS, DH), jnp.float32),
        mesh=_mesh,
        compiler_params=_sc_params_b,
        scratch_types=[
            pltpu.VMEM((NROWS,), jnp.float32),
            pltpu.VMEM((NROWS,), jnp.float32),
            pltpu.VMEM((RPT, 128), jnp.int32),
            pltpu.VMEM((RPT, 128), jnp.int32),
            pltpu.VMEM((RPT, 128), jnp.int32),
            pltpu.VMEM((128, DH), jnp.float32),
            pltpu.VMEM((128, DH), jnp.float32),
            pltpu.VMEM((128, DH), jnp.float32),
            pltpu.VMEM((128, DH), jnp.float32),
            pltpu.VMEM((RPT, 128), jnp.float32),
            pltpu.VMEM((RPT, 128), jnp.float32),
            pltpu.SemaphoreType.DMA,
            pltpu.SemaphoreType.DMA,
            pltpu.SemaphoreType.DMA,
            pltpu.SemaphoreType.DMA,
            pltpu.VMEM_SHARED((NROWS, DH), jnp.float32),
        ],
    )(hAh, hroth, ex, s_tot, idxs)


def _bc_out_kernel(hb_ref, hc_ref, wo_ref, bo_ref, outB_ref, outC_ref):
    wo = wo_ref[...]
    bo = bo_ref[...]
    outB_ref[...] = jnp.dot(hb_ref[...], wo, preferred_element_type=jnp.float32) + bo
    outC_ref[...] = jnp.dot(hc_ref[...], wo, preferred_element_type=jnp.float32) + bo


def _bc_out(hB, hC, W_out, b_out):
    grid = NA // ROW_BLK
    return pl.pallas_call(
        _bc_out_kernel,
        grid=(grid,),
        in_specs=[
            pl.BlockSpec((ROW_BLK, D), lambda i: (i, 0)),
            pl.BlockSpec((ROW_BLK, D), lambda i: (i, 0)),
            pl.BlockSpec((D, OUT), lambda i: (0, 0)),
            pl.BlockSpec((1, OUT), lambda i: (0, 0)),
        ],
        out_specs=[
            pl.BlockSpec((ROW_BLK, OUT), lambda i: (i, 0)),
            pl.BlockSpec((ROW_BLK, OUT), lambda i: (i, 0)),
        ],
        out_shape=[
            jax.ShapeDtypeStruct((NA, OUT), jnp.float32),
            jax.ShapeDtypeStruct((NA, OUT), jnp.float32),
        ],
    )(hB, hC, W_out, b_out.reshape(1, OUT))


def _tail_reduce_kernel(acc_ref, ha_ref, has_ref, wl_ref, bl_ref,
                        out_ref, fa1_ref, fa2_ref):
    i = pl.program_id(0)
    ha = ha_ref[...]
    fas = []
    for mp in range(2):
        h0 = acc_ref[mp, 0, 0] + acc_ref[mp, 0, 1]
        h1 = acc_ref[mp, 1, 0] + acc_ref[mp, 1, 1]
        h = jnp.concatenate([h0, h1], axis=-1)
        h = (h + ha * has_ref[:, mp:mp + 1]) * (1.0 / 3.0)
        fas.append(jnp.where(h > 0.0, h, jnp.exp(h) - 1.0))
    fa1_ref[...] = fas[0]
    fa2_ref[...] = fas[1]
    t1 = jnp.tanh(jnp.dot(fas[0], wl_ref[...],
                          preferred_element_type=jnp.float32) + bl_ref[...])
    t2 = jnp.tanh(jnp.dot(fas[1], wl_ref[...],
                          preferred_element_type=jnp.float32) + bl_ref[...])
    part = jnp.stack([jnp.sum(t1, axis=0), jnp.sum(t2, axis=0)], axis=0)

    @pl.when(i == 0)
    def _init():
        out_ref[...] = jnp.zeros_like(out_ref)

    out_ref[...] += part


def _tail_out_kernel(beta_ref, fa1_ref, fa2_ref, wo_ref,
                     bo_ref, featA_ref, outA_ref):
    b0 = beta_ref[0]
    b1 = beta_ref[1]
    featA = b0 * fa1_ref[...] + b1 * fa2_ref[...]
    featA_ref[...] = featA
    outA_ref[...] = jnp.dot(featA, wo_ref[...],
                            preferred_element_type=jnp.float32) + bo_ref[...]


def _tail(acc, hA, has, Wl, bl, v, W_out, b_out):
    grid = NA // ROW_BLK
    tsum, fa1, fa2 = pl.pallas_call(
        _tail_reduce_kernel,
        grid=(grid,),
        in_specs=[
            pl.BlockSpec((2, 2, NC, ROW_BLK, DH), lambda i: (0, 0, 0, i, 0)),
            pl.BlockSpec((ROW_BLK, D), lambda i: (i, 0)),
            pl.BlockSpec((ROW_BLK, 2), lambda i: (i, 0)),
            pl.BlockSpec((D, ATTN), lambda i: (0, 0)),
            pl.BlockSpec((1, ATTN), lambda i: (0, 0)),
        ],
        out_specs=[
            pl.BlockSpec((2, ATTN), lambda i: (0, 0)),
            pl.BlockSpec((ROW_BLK, D), lambda i: (i, 0)),
            pl.BlockSpec((ROW_BLK, D), lambda i: (i, 0)),
        ],
        out_shape=[
            jax.ShapeDtypeStruct((2, ATTN), jnp.float32),
            jax.ShapeDtypeStruct((NA, D), jnp.float32),
            jax.ShapeDtypeStruct((NA, D), jnp.float32),
        ],
    )(acc, hA, has, Wl, bl.reshape(1, ATTN))
    scores = (tsum / NA) @ v
    beta = jax.nn.softmax(scores)
    featA, outA = pl.pallas_call(
        _tail_out_kernel,
        grid=(grid,),
        in_specs=[
            pl.BlockSpec(memory_space=pltpu.SMEM),
            pl.BlockSpec((ROW_BLK, D), lambda i: (i, 0)),
            pl.BlockSpec((ROW_BLK, D), lambda i: (i, 0)),
            pl.BlockSpec((D, OUT), lambda i: (0, 0)),
            pl.BlockSpec((1, OUT), lambda i: (0, 0)),
        ],
        out_specs=[
            pl.BlockSpec((ROW_BLK, D), lambda i: (i, 0)),
            pl.BlockSpec((ROW_BLK, OUT), lambda i: (i, 0)),
        ],
        out_shape=[
            jax.ShapeDtypeStruct((NA, D), jnp.float32),
            jax.ShapeDtypeStruct((NA, OUT), jnp.float32),
        ],
    )(beta, fa1, fa2, W_out, b_out.reshape(1, OUT))
    return featA, outA


def _rotate(h, r):
    hc = h.reshape(-1, D // 2, 2)
    hr, hi = hc[:, :, 0], hc[:, :, 1]
    rr, ri = r[:, 0], r[:, 1]
    return jnp.stack([hr * rr - hi * ri, hr * ri + hi * rr], axis=2).reshape(-1, D)


def _pad_cols(idx):
    npad = NI_PAD - NI
    j = jnp.arange(npad, dtype=jnp.int32)
    i0 = jnp.concatenate([idx[:, 0].astype(jnp.int32), NA + (j % NJUNK)])
    i1 = jnp.concatenate([idx[:, 1].astype(jnp.int32), j % NA])
    i2 = jnp.concatenate([idx[:, 2].astype(jnp.int32), j % NA])
    return (i0.reshape(IROWS, 128), i1.reshape(IROWS, 128),
            i2.reshape(IROWS, 128))


def _ptab(p):
    return jnp.concatenate(
        [p.reshape(NA * H), jnp.zeros((PTAB - NA * H,), jnp.float32)])


def kernel(hA, hB, hC, idx_ABA, idx_ACA, attn_r_ABA, attn_r_ACA,
           rAB, rBA, rAC, rCA, Wl, bl, v, W_out, b_out):
    outB, outC = _bc_out(hB, hC, W_out, b_out)
    hBrot = _rotate(hB, rAB)
    hCrot = _rotate(hC, rAC)
    hA4 = hA.reshape(NA, H, F)
    pa1 = jnp.einsum("nhf,hf->nh", hA4, attn_r_ABA[0])
    pb1 = jnp.einsum("nhf,hf->nh", hBrot.reshape(NA, H, F), attn_r_ABA[0])
    pa2 = jnp.einsum("nhf,hf->nh", hA4, attn_r_ACA[0])
    pc2 = jnp.einsum("nhf,hf->nh", hCrot.reshape(NA, H, F), attn_r_ACA[0])

    tab = jnp.stack([jnp.stack([_ptab(pa1), _ptab(pb1)]),
                     jnp.stack([_ptab(pa2), _ptab(pc2)])])
    idxs = jnp.stack([jnp.stack(_pad_cols(idx_ABA)),
                      jnp.stack(_pad_cols(idx_ACA))])

    s_pair, ex = _pass_a(tab, idxs)
    s_tot = s_pair[:, 0] + s_pair[:, 1]
    hAh = jnp.stack([hA[:, :DH], hA[:, DH:]])
    hroth = jnp.stack([hBrot[:, :DH], hBrot[:, DH:],
                       hCrot[:, :DH], hCrot[:, DH:]])
    acc = _pass_b(hAh, hroth, ex, s_tot, idxs)
    has = (s_tot[:, 0] > 0.0).astype(jnp.float32).T

    featA, outA = _tail(acc, hA, has, Wl, bl, v, W_out, b_out)
    return (outA, outB, outC, featA, hB, hC)


# speedup vs baseline: 1.0539x; 1.0008x over previous
import functools

import jax
import jax.numpy as jnp
from jax import lax
from jax.experimental import pallas as pl
from jax.experimental.pallas import tpu as pltpu
from jax.experimental.pallas import tpu_sc as plsc

NA = 10000
NI = 160000
D = 128
H = 4
F = 32
OUT = 256
ATTN = 128

NC = 2
NS = 16
NW = NC * NS

NI_PAD = 163840
EPT = NI_PAD // NW
ECH = 1024
NCH_A = EPT // ECH
KCH = ECH // 128
IROWS = NI_PAD // 128
RPT = IROWS // NW
NROWS = 10752
NJUNK = 512
ZPT = NROWS // NS
PTAB = 4 * NROWS

ROW_BLK = 1000

_mesh = plsc.VectorSubcoreMesh(core_axis_name="c", subcore_axis_name="s")
_sc_params = pltpu.CompilerParams(needs_layout_passes=False)
_sc_params_b = pltpu.CompilerParams(needs_layout_passes=False,
                                    use_tc_tiling_on_sc=False)


def _pass_a_body(tab_hbm, idx_hbm, s_out, ex_out,
                 pa_v, pb_v, i0b, i1b, i2b, exb0, exb1, exb2, exb3, zb,
                 s0, s1, s2, s3):
    cid = lax.axis_index("c")
    sid = lax.axis_index("s")
    wid = sid * NC + cid
    exbs = (exb0, exb1, exb2, exb3)
    ssh = (s0, s1, s2, s3)

    for t in range(ZPT // 16):
        zb[pl.ds(t * 16, 16)] = jnp.zeros((16,), jnp.float32)

    def mp_iter(mp, _):
        pltpu.sync_copy(tab_hbm.at[mp, 0], pa_v)
        pltpu.sync_copy(tab_hbm.at[mp, 1], pb_v)
        for s_sh in ssh:
            pltpu.sync_copy(zb, s_sh.at[pl.ds(sid * ZPT, ZPT)])
        plsc.subcore_barrier()

        def chunk(c, _):
            rb = wid * RPT + c * KCH
            pltpu.sync_copy(idx_hbm.at[mp, 0, pl.ds(rb, KCH)], i0b)
            pltpu.sync_copy(idx_hbm.at[mp, 1, pl.ds(rb, KCH)], i1b)
            pltpu.sync_copy(idx_hbm.at[mp, 2, pl.ds(rb, KCH)], i2b)

            def grp(g, _):
                k = g // 8
                m = g % 8
                off = m * 16
                b0 = i0b[k, pl.ds(off, 16)] * 4
                b1 = i1b[k, pl.ds(off, 16)] * 4
                b2 = i2b[k, pl.ds(off, 16)] * 4
                for h in range(H):
                    er = (plsc.load_gather(pa_v, [b0 + h])
                          + plsc.load_gather(pb_v, [b1 + h])
                          + plsc.load_gather(pa_v, [b2 + h])) * (1.0 / 3.0)
                    e = jnp.maximum(er, 0.01 * er)
                    exbs[h][k, pl.ds(off, 16)] = jnp.exp(e)
                return _

            lax.fori_loop(0, ECH // 16, grp, None)
            for h in range(H):
                pltpu.sync_copy(exbs[h], ex_out.at[mp, h, pl.ds(rb, KCH)])
            for k in range(KCH):
                for h in range(H):
                    pltpu.sync_copy(exbs[h].at[k], ssh[h].at[i0b.at[k]],
                                    add=True)
            return _

        lax.fori_loop(0, NCH_A, chunk, None)
        plsc.subcore_barrier()

        @pl.when(sid == 0)
        def _copy_out():
            for h in range(H):
                pltpu.sync_copy(ssh[h], s_out.at[mp, cid, h])

        plsc.subcore_barrier()
        return _

    lax.fori_loop(0, 2, mp_iter, None)


def _pass_a(tab, idxs):
    return pl.kernel(
        _pass_a_body,
        out_type=(
            jax.ShapeDtypeStruct((2, NC, H, NROWS), jnp.float32),
            jax.ShapeDtypeStruct((2, H, IROWS, 128), jnp.float32),
        ),
        mesh=_mesh,
        compiler_params=_sc_params,
        scratch_types=[
            pltpu.VMEM((PTAB,), jnp.float32),
            pltpu.VMEM((PTAB,), jnp.float32),
            pltpu.VMEM((KCH, 128), jnp.int32),
            pltpu.VMEM((KCH, 128), jnp.int32),
            pltpu.VMEM((KCH, 128), jnp.int32),
            pltpu.VMEM((KCH, 128), jnp.float32),
            pltpu.VMEM((KCH, 128), jnp.float32),
            pltpu.VMEM((KCH, 128), jnp.float32),
            pltpu.VMEM((KCH, 128), jnp.float32),
            pltpu.VMEM((ZPT,), jnp.float32),
            pltpu.VMEM_SHARED((NROWS,), jnp.float32),
            pltpu.VMEM_SHARED((NROWS,), jnp.float32),
            pltpu.VMEM_SHARED((NROWS,), jnp.float32),
            pltpu.VMEM_SHARED((NROWS,), jnp.float32),
        ],
    )(tab, idxs)


DH = D // 2


def _pass_b_body(hA_hbm, hrot_hbm, ex_hbm, s_hbm, idx_hbm, acc_out,
                 s0v, s1v, i0b, i1b, i2b, rA, rB,
                 wbuf0, wbuf1, exb0, exb1,
                 semA, semB, semS0, semS1, acc):
    cid = lax.axis_index("c")
    sid = lax.axis_index("s")
    wid = sid * NC + cid
    svs = (s0v, s1v)
    wbufs = (wbuf0, wbuf1)
    exbs = (exb0, exb1)

    def q_iter(q, _):
        mp = q // 2
        half = q % 2

        def zrow(t, _):
            for j in range(DH // 16):
                wbuf0[t, pl.ds(j * 16, 16)] = jnp.zeros((16,), jnp.float32)
                wbuf1[t, pl.ds(j * 16, 16)] = jnp.zeros((16,), jnp.float32)
            return _

        lax.fori_loop(0, 128, zrow, None)
        for t in range(5):
            pltpu.sync_copy(wbuf0, acc.at[pl.ds(sid * ZPT + t * 128, 128)])
        pltpu.sync_copy(wbuf0.at[pl.ds(0, 32)],
                        acc.at[pl.ds(sid * ZPT + 640, 32)])
        for hh in range(2):
            pltpu.sync_copy(s_hbm.at[mp, half * 2 + hh], svs[hh])
        pltpu.sync_copy(idx_hbm.at[mp, 0, pl.ds(wid * RPT, RPT)], i0b)
        pltpu.sync_copy(idx_hbm.at[mp, 1, pl.ds(wid * RPT, RPT)], i1b)
        pltpu.sync_copy(idx_hbm.at[mp, 2, pl.ds(wid * RPT, RPT)], i2b)
        for hh in range(2):
            pltpu.sync_copy(
                ex_hbm.at[mp, half * 2 + hh, pl.ds(wid * RPT, RPT)], exbs[hh])
        plsc.subcore_barrier()
        semSs = (semS0, semS1)
        for p in range(2):
            pltpu.async_copy(wbufs[p], acc.at[i0b.at[0]], semSs[p], add=True)

        def do_chunk(c, p):
            cpB = pltpu.async_copy(hrot_hbm.at[mp * 2 + half].at[i1b.at[c]],
                                   rB, semB)
            cpA = pltpu.async_copy(hA_hbm.at[half].at[i2b.at[c]], rA, semA)

            @plsc.parallel_loop(0, 8)
            def agrp(m):
                off = m * 16
                i0v = i0b[c, pl.ds(off, 16)]
                for hh in range(2):
                    sv = plsc.load_gather(svs[hh], [i0v])
                    exbs[hh][c, pl.ds(off, 16)] = (
                        exbs[hh][c, pl.ds(off, 16)] / sv)

            pltpu.make_async_copy(wbufs[p], acc.at[i0b.at[c]],
                                  semSs[p]).wait()
            cpB.wait()
            cpA.wait()
            cv = jnp.full((16,), c, jnp.int32)
            wb = wbufs[p]

            @plsc.parallel_loop(0, 128, unroll=8)
            def edge(e):
                ev = jnp.full((16,), e, jnp.int32)
                sp = [plsc.load_gather(exbs[hh], [cv, ev]) for hh in range(2)]
                for j in range(DH // 16):
                    w = (rB[e, pl.ds(j * 16, 16)]
                         + rA[e, pl.ds(j * 16, 16)]) * sp[j // 2]
                    wb[e, pl.ds(j * 16, 16)] = w

            pltpu.async_copy(wb, acc.at[i0b.at[c]], semSs[p], add=True)

        def chunk_pair(t, _):
            for p in range(2):
                do_chunk(2 * t + p, p)
            return _

        lax.fori_loop(0, RPT // 2, chunk_pair, None)
        for p in range(2):
            pltpu.make_async_copy(wbufs[p], acc.at[i0b.at[0]], semSs[p]).wait()
        plsc.subcore_barrier()
        pltpu.sync_copy(acc.at[pl.ds(sid * ZPT, ZPT)],
                        acc_out.at[mp, half, cid, pl.ds(sid * ZPT, ZPT)])
        plsc.subcore_barrier()
        return _

    lax.fori_loop(0, 4, q_iter, None)


def _pass_b(hAh, hroth, ex, s_tot, idxs):
    return pl.kernel(
        _pass_b_body,
        out_type=jax.ShapeDtypeStruct((2, 2, NC, NROWS, DH), jnp.float32),
        mesh=_mesh,
        compiler_params=_sc_params_b,
        scratch_types=[
            pltpu.VMEM((NROWS,), jnp.float32),
            pltpu.VMEM((NROWS,), jnp.float32),
            pltpu.VMEM((RPT, 128), jnp.int32),
            pltpu.VMEM((RPT, 128), jnp.int32),
            pltpu.VMEM((RPT, 128), jnp.int32),
            pltpu.VMEM((128, DH), jnp.float32),
            pltpu.VMEM((128, DH), jnp.float32),
            pltpu.VMEM((128, DH), jnp.float32),
            pltpu.VMEM((128, DH), jnp.float32),
            pltpu.VMEM((RPT, 128), jnp.float32),
            pltpu.VMEM((RPT, 128), jnp.float32),
            pltpu.SemaphoreType.DMA,
            pltpu.SemaphoreType.DMA,
            pltpu.SemaphoreType.DMA,
            pltpu.SemaphoreType.DMA,
            pltpu.VMEM_SHARED((NROWS, DH), jnp.float32),
        ],
    )(hAh, hroth, ex, s_tot, idxs)


def _bc_out_kernel(hb_ref, hc_ref, wo_ref, bo_ref, outB_ref, outC_ref):
    wo = wo_ref[...]
    bo = bo_ref[...]
    outB_ref[...] = jnp.dot(hb_ref[...], wo, preferred_element_type=jnp.float32) + bo
    outC_ref[...] = jnp.dot(hc_ref[...], wo, preferred_element_type=jnp.float32) + bo


def _bc_out(hB, hC, W_out, b_out):
    grid = NA // ROW_BLK
    return pl.pallas_call(
        _bc_out_kernel,
        grid=(grid,),
        in_specs=[
            pl.BlockSpec((ROW_BLK, D), lambda i: (i, 0)),
            pl.BlockSpec((ROW_BLK, D), lambda i: (i, 0)),
            pl.BlockSpec((D, OUT), lambda i: (0, 0)),
            pl.BlockSpec((1, OUT), lambda i: (0, 0)),
        ],
        out_specs=[
            pl.BlockSpec((ROW_BLK, OUT), lambda i: (i, 0)),
            pl.BlockSpec((ROW_BLK, OUT), lambda i: (i, 0)),
        ],
        out_shape=[
            jax.ShapeDtypeStruct((NA, OUT), jnp.float32),
            jax.ShapeDtypeStruct((NA, OUT), jnp.float32),
        ],
    )(hB, hC, W_out, b_out.reshape(1, OUT))


def _tail_reduce_kernel(acc_ref, ha_ref, has_ref, wl_ref, bl_ref,
                        out_ref, fa1_ref, fa2_ref):
    i = pl.program_id(0)
    ha = ha_ref[...]
    fas = []
    for mp in range(2):
        h0 = acc_ref[mp, 0, 0] + acc_ref[mp, 0, 1]
        h1 = acc_ref[mp, 1, 0] + acc_ref[mp, 1, 1]
        h = jnp.concatenate([h0, h1], axis=-1)
        h = (h + ha * has_ref[:, mp:mp + 1]) * (1.0 / 3.0)
        fas.append(jnp.where(h > 0.0, h, jnp.exp(h) - 1.0))
    fa1_ref[...] = fas[0]
    fa2_ref[...] = fas[1]
    t1 = jnp.tanh(jnp.dot(fas[0], wl_ref[...],
                          preferred_element_type=jnp.float32) + bl_ref[...])
    t2 = jnp.tanh(jnp.dot(fas[1], wl_ref[...],
                          preferred_element_type=jnp.float32) + bl_ref[...])
    part = jnp.stack([jnp.sum(t1, axis=0), jnp.sum(t2, axis=0)], axis=0)

    @pl.when(i == 0)
    def _init():
        out_ref[...] = jnp.zeros_like(out_ref)

    out_ref[...] += part


def _tail_out_kernel(beta_ref, fa1_ref, fa2_ref, wo_ref,
                     bo_ref, featA_ref, outA_ref):
    b0 = beta_ref[0]
    b1 = beta_ref[1]
    featA = b0 * fa1_ref[...] + b1 * fa2_ref[...]
    featA_ref[...] = featA
    outA_ref[...] = jnp.dot(featA, wo_ref[...],
                            preferred_element_type=jnp.float32) + bo_ref[...]


def _tail(acc, hA, has, Wl, bl, v, W_out, b_out):
    grid = NA // ROW_BLK
    tsum, fa1, fa2 = pl.pallas_call(
        _tail_reduce_kernel,
        grid=(grid,),
        in_specs=[
            pl.BlockSpec((2, 2, NC, ROW_BLK, DH), lambda i: (0, 0, 0, i, 0)),
            pl.BlockSpec((ROW_BLK, D), lambda i: (i, 0)),
            pl.BlockSpec((ROW_BLK, 2), lambda i: (i, 0)),
            pl.BlockSpec((D, ATTN), lambda i: (0, 0)),
            pl.BlockSpec((1, ATTN), lambda i: (0, 0)),
        ],
        out_specs=[
            pl.BlockSpec((2, ATTN), lambda i: (0, 0)),
            pl.BlockSpec((ROW_BLK, D), lambda i: (i, 0)),
            pl.BlockSpec((ROW_BLK, D), lambda i: (i, 0)),
        ],
        out_shape=[
            jax.ShapeDtypeStruct((2, ATTN), jnp.float32),
            jax.ShapeDtypeStruct((NA, D), jnp.float32),
            jax.ShapeDtypeStruct((NA, D), jnp.float32),
        ],
    )(acc, hA, has, Wl, bl.reshape(1, ATTN))
    scores = (tsum / NA) @ v
    beta = jax.nn.softmax(scores)
    featA, outA = pl.pallas_call(
        _tail_out_kernel,
        grid=(grid,),
        in_specs=[
            pl.BlockSpec(memory_space=pltpu.SMEM),
            pl.BlockSpec((ROW_BLK, D), lambda i: (i, 0)),
            pl.BlockSpec((ROW_BLK, D), lambda i: (i, 0)),
            pl.BlockSpec((D, OUT), lambda i: (0, 0)),
            pl.BlockSpec((1, OUT), lambda i: (0, 0)),
        ],
        out_specs=[
            pl.BlockSpec((ROW_BLK, D), lambda i: (i, 0)),
            pl.BlockSpec((ROW_BLK, OUT), lambda i: (i, 0)),
        ],
        out_shape=[
            jax.ShapeDtypeStruct((NA, D), jnp.float32),
            jax.ShapeDtypeStruct((NA, OUT), jnp.float32),
        ],
    )(beta, fa1, fa2, W_out, b_out.reshape(1, OUT))
    return featA, outA


def _rotate(h, r):
    hc = h.reshape(-1, D // 2, 2)
    hr, hi = hc[:, :, 0], hc[:, :, 1]
    rr, ri = r[:, 0], r[:, 1]
    return jnp.stack([hr * rr - hi * ri, hr * ri + hi * rr], axis=2).reshape(-1, D)


def _pad_cols(idx):
    npad = NI_PAD - NI
    j = jnp.arange(npad, dtype=jnp.int32)
    i0 = jnp.concatenate([idx[:, 0].astype(jnp.int32), NA + (j % NJUNK)])
    i1 = jnp.concatenate([idx[:, 1].astype(jnp.int32), j % NA])
    i2 = jnp.concatenate([idx[:, 2].astype(jnp.int32), j % NA])
    return (i0.reshape(IROWS, 128), i1.reshape(IROWS, 128),
            i2.reshape(IROWS, 128))


def _ptab(p):
    return jnp.concatenate(
        [p.reshape(NA * H), jnp.zeros((PTAB - NA * H,), jnp.float32)])


def kernel(hA, hB, hC, idx_ABA, idx_ACA, attn_r_ABA, attn_r_ACA,
           rAB, rBA, rAC, rCA, Wl, bl, v, W_out, b_out):
    outB, outC = _bc_out(hB, hC, W_out, b_out)
    hBrot = _rotate(hB, rAB)
    hCrot = _rotate(hC, rAC)
    hA4 = hA.reshape(NA, H, F)
    pa1 = jnp.einsum("nhf,hf->nh", hA4, attn_r_ABA[0])
    pb1 = jnp.einsum("nhf,hf->nh", hBrot.reshape(NA, H, F), attn_r_ABA[0])
    pa2 = jnp.einsum("nhf,hf->nh", hA4, attn_r_ACA[0])
    pc2 = jnp.einsum("nhf,hf->nh", hCrot.reshape(NA, H, F), attn_r_ACA[0])

    tab = jnp.stack([jnp.stack([_ptab(pa1), _ptab(pb1)]),
                     jnp.stack([_ptab(pa2), _ptab(pc2)])])
    idxs = jnp.stack([jnp.stack(_pad_cols(idx_ABA)),
                      jnp.stack(_pad_cols(idx_ACA))])

    s_pair, ex = _pass_a(tab, idxs)
    s_tot = s_pair[:, 0] + s_pair[:, 1]
    hAh = jnp.stack([hA[:, :DH], hA[:, DH:]])
    hroth = jnp.stack([hBrot[:, :DH], hBrot[:, DH:],
                       hCrot[:, :DH], hCrot[:, DH:]])
    acc = _pass_b(hAh, hroth, ex, s_tot, idxs)
    has = (s_tot[:, 0] > 0.0).astype(jnp.float32).T

    featA, outA = _tail(acc, hA, has, Wl, bl, v, W_out, b_out)
    return (outA, outB, outC, featA, hB, hC)


# speedup vs baseline: 1.0547x; 1.0008x over previous
import jax
import jax.numpy as jnp
from jax import lax
from jax.experimental import pallas as pl
from jax.experimental.pallas import tpu as pltpu
from jax.experimental.pallas import tpu_sc as plsc

NA = 10000
NI = 160000
D = 128
H = 4
F = 32
OUT = 256
ATTN = 128

NC = 2
NS = 16
NW = NC * NS

NI_PAD = 163840
EPT = NI_PAD // NW
ECH = 1024
NCH_A = EPT // ECH
KCH = ECH // 128
IROWS = NI_PAD // 128
RPT = IROWS // NW
NROWS = 10752
NJUNK = 512
ZPT = NROWS // NS
PTAB = 4 * NROWS

ROW_BLK = 1000

_mesh = plsc.VectorSubcoreMesh(core_axis_name="c", subcore_axis_name="s")
_sc_params = pltpu.CompilerParams(needs_layout_passes=False)
_sc_params_b = pltpu.CompilerParams(needs_layout_passes=False,
                                    use_tc_tiling_on_sc=False)


def _pass_a_body(tab_hbm, idx_hbm, s_out, ex_out,
                 pa_v, pb_v, i0b, i1b, i2b, exb0, exb1, exb2, exb3, zb,
                 s0, s1, s2, s3):
    cid = lax.axis_index("c")
    sid = lax.axis_index("s")
    wid = sid * NC + cid
    exbs = (exb0, exb1, exb2, exb3)
    ssh = (s0, s1, s2, s3)

    for t in range(ZPT // 16):
        zb[pl.ds(t * 16, 16)] = jnp.zeros((16,), jnp.float32)

    def mp_iter(mp, _):
        pltpu.sync_copy(tab_hbm.at[mp, 0], pa_v)
        pltpu.sync_copy(tab_hbm.at[mp, 1], pb_v)
        for s_sh in ssh:
            pltpu.sync_copy(zb, s_sh.at[pl.ds(sid * ZPT, ZPT)])
        plsc.subcore_barrier()

        def chunk(c, _):
            rb = wid * RPT + c * KCH
            pltpu.sync_copy(idx_hbm.at[mp, 0, pl.ds(rb, KCH)], i0b)
            pltpu.sync_copy(idx_hbm.at[mp, 1, pl.ds(rb, KCH)], i1b)
            pltpu.sync_copy(idx_hbm.at[mp, 2, pl.ds(rb, KCH)], i2b)

            def grp(g, _):
                k = g // 8
                m = g % 8
                off = m * 16
                b0 = i0b[k, pl.ds(off, 16)] * 4
                b1 = i1b[k, pl.ds(off, 16)] * 4
                b2 = i2b[k, pl.ds(off, 16)] * 4
                for h in range(H):
                    er = (plsc.load_gather(pa_v, [b0 + h])
                          + plsc.load_gather(pb_v, [b1 + h])
                          + plsc.load_gather(pa_v, [b2 + h])) * (1.0 / 3.0)
                    e = jnp.maximum(er, 0.01 * er)
                    exbs[h][k, pl.ds(off, 16)] = jnp.exp(e)
                return _

            lax.fori_loop(0, ECH // 16, grp, None)
            for h in range(H):
                pltpu.sync_copy(exbs[h], ex_out.at[mp, h, pl.ds(rb, KCH)])
            for k in range(KCH):
                for h in range(H):
                    pltpu.sync_copy(exbs[h].at[k], ssh[h].at[i0b.at[k]],
                                    add=True)
            return _

        lax.fori_loop(0, NCH_A, chunk, None)
        plsc.subcore_barrier()

        @pl.when(sid == 0)
        def _copy_out():
            for h in range(H):
                pltpu.sync_copy(ssh[h], s_out.at[mp, cid, h])

        plsc.subcore_barrier()
        return _

    lax.fori_loop(0, 2, mp_iter, None)


def _pass_a(tab, idxs):
    return pl.kernel(
        _pass_a_body,
        out_type=(
            jax.ShapeDtypeStruct((2, NC, H, NROWS), jnp.float32),
            jax.ShapeDtypeStruct((2, H, IROWS, 128), jnp.float32),
        ),
        mesh=_mesh,
        compiler_params=_sc_params,
        scratch_types=[
            pltpu.VMEM((PTAB,), jnp.float32),
            pltpu.VMEM((PTAB,), jnp.float32),
            pltpu.VMEM((KCH, 128), jnp.int32),
            pltpu.VMEM((KCH, 128), jnp.int32),
            pltpu.VMEM((KCH, 128), jnp.int32),
            pltpu.VMEM((KCH, 128), jnp.float32),
            pltpu.VMEM((KCH, 128), jnp.float32),
            pltpu.VMEM((KCH, 128), jnp.float32),
            pltpu.VMEM((KCH, 128), jnp.float32),
            pltpu.VMEM((ZPT,), jnp.float32),
            pltpu.VMEM_SHARED((NROWS,), jnp.float32),
            pltpu.VMEM_SHARED((NROWS,), jnp.float32),
            pltpu.VMEM_SHARED((NROWS,), jnp.float32),
            pltpu.VMEM_SHARED((NROWS,), jnp.float32),
        ],
    )(tab, idxs)


DH = D // 2


def _pass_b_body(hA_hbm, hrot_hbm, ex_hbm, s_hbm, idx_hbm, acc_out,
                 s0v, s1v, i0b, i1b, i2b, rA, rB,
                 wbuf0, wbuf1, exb0, exb1,
                 semA, semB, semS0, semS1, acc):
    cid = lax.axis_index("c")
    sid = lax.axis_index("s")
    wid = sid * NC + cid
    svs = (s0v, s1v)
    wbufs = (wbuf0, wbuf1)
    exbs = (exb0, exb1)

    def q_iter(q, _):
        mp = q // 2
        half = q % 2

        def zrow(t, _):
            for j in range(DH // 16):
                wbuf0[t, pl.ds(j * 16, 16)] = jnp.zeros((16,), jnp.float32)
                wbuf1[t, pl.ds(j * 16, 16)] = jnp.zeros((16,), jnp.float32)
            return _

        lax.fori_loop(0, 128, zrow, None)
        for t in range(5):
            pltpu.sync_copy(wbuf0, acc.at[pl.ds(sid * ZPT + t * 128, 128)])
        pltpu.sync_copy(wbuf0.at[pl.ds(0, 32)],
                        acc.at[pl.ds(sid * ZPT + 640, 32)])
        for hh in range(2):
            pltpu.sync_copy(s_hbm.at[mp, half * 2 + hh], svs[hh])
        pltpu.sync_copy(idx_hbm.at[mp, 0, pl.ds(wid * RPT, RPT)], i0b)
        pltpu.sync_copy(idx_hbm.at[mp, 1, pl.ds(wid * RPT, RPT)], i1b)
        pltpu.sync_copy(idx_hbm.at[mp, 2, pl.ds(wid * RPT, RPT)], i2b)
        for hh in range(2):
            pltpu.sync_copy(
                ex_hbm.at[mp, half * 2 + hh, pl.ds(wid * RPT, RPT)], exbs[hh])
        plsc.subcore_barrier()
        semSs = (semS0, semS1)
        for p in range(2):
            pltpu.async_copy(wbufs[p], acc.at[i0b.at[0]], semSs[p], add=True)

        def do_chunk(c, p):
            cpB = pltpu.async_copy(hrot_hbm.at[mp * 2 + half].at[i1b.at[c]],
                                   rB, semB)
            cpA = pltpu.async_copy(hA_hbm.at[half].at[i2b.at[c]], rA, semA)

            @plsc.parallel_loop(0, 8)
            def agrp(m):
                off = m * 16
                i0v = i0b[c, pl.ds(off, 16)]
                for hh in range(2):
                    sv = plsc.load_gather(svs[hh], [i0v])
                    exbs[hh][c, pl.ds(off, 16)] = (
                        exbs[hh][c, pl.ds(off, 16)] / sv)

            pltpu.make_async_copy(wbufs[p], acc.at[i0b.at[c]],
                                  semSs[p]).wait()
            cpB.wait()
            cpA.wait()
            cv = jnp.full((16,), c, jnp.int32)
            wb = wbufs[p]

            @plsc.parallel_loop(0, 128, unroll=8)
            def edge(e):
                ev = jnp.full((16,), e, jnp.int32)
                sp = [plsc.load_gather(exbs[hh], [cv, ev]) for hh in range(2)]
                for j in range(DH // 16):
                    w = (rB[e, pl.ds(j * 16, 16)]
                         + rA[e, pl.ds(j * 16, 16)]) * sp[j // 2]
                    wb[e, pl.ds(j * 16, 16)] = w

            pltpu.async_copy(wb, acc.at[i0b.at[c]], semSs[p], add=True)

        def chunk_pair(t, _):
            for p in range(2):
                do_chunk(2 * t + p, p)
            return _

        lax.fori_loop(0, RPT // 2, chunk_pair, None)
        for p in range(2):
            pltpu.make_async_copy(wbufs[p], acc.at[i0b.at[0]], semSs[p]).wait()
        plsc.subcore_barrier()
        pltpu.sync_copy(acc.at[pl.ds(sid * ZPT, ZPT)],
                        acc_out.at[mp, half, cid, pl.ds(sid * ZPT, ZPT)])
        plsc.subcore_barrier()
        return _

    lax.fori_loop(0, 4, q_iter, None)


def _pass_b(hAh, hroth, ex, s_tot, idxs):
    return pl.kernel(
        _pass_b_body,
        out_type=jax.ShapeDtypeStruct((2, 2, NC, NROWS, DH), jnp.float32),
        mesh=_mesh,
        compiler_params=_sc_params_b,
        scratch_types=[
            pltpu.VMEM((NROWS,), jnp.float32),
            pltpu.VMEM((NROWS,), jnp.float32),
            pltpu.VMEM((RPT, 128), jnp.int32),
            pltpu.VMEM((RPT, 128), jnp.int32),
            pltpu.VMEM((RPT, 128), jnp.int32),
            pltpu.VMEM((128, DH), jnp.float32),
            pltpu.VMEM((128, DH), jnp.float32),
            pltpu.VMEM((128, DH), jnp.float32),
            pltpu.VMEM((128, DH), jnp.float32),
            pltpu.VMEM((RPT, 128), jnp.float32),
            pltpu.VMEM((RPT, 128), jnp.float32),
            pltpu.SemaphoreType.DMA,
            pltpu.SemaphoreType.DMA,
            pltpu.SemaphoreType.DMA,
            pltpu.SemaphoreType.DMA,
            pltpu.VMEM_SHARED((NROWS, DH), jnp.float32),
        ],
    )(hAh, hroth, ex, s_tot, idxs)


def _bc_out_kernel(hb_ref, hc_ref, wo_ref, bo_ref, outB_ref, outC_ref):
    wo = wo_ref[...]
    bo = bo_ref[...]
    outB_ref[...] = jnp.dot(hb_ref[...], wo, preferred_element_type=jnp.float32) + bo
    outC_ref[...] = jnp.dot(hc_ref[...], wo, preferred_element_type=jnp.float32) + bo


def _bc_out(hB, hC, W_out, b_out):
    grid = NA // ROW_BLK
    return pl.pallas_call(
        _bc_out_kernel,
        grid=(grid,),
        in_specs=[
            pl.BlockSpec((ROW_BLK, D), lambda i: (i, 0)),
            pl.BlockSpec((ROW_BLK, D), lambda i: (i, 0)),
            pl.BlockSpec((D, OUT), lambda i: (0, 0)),
            pl.BlockSpec((1, OUT), lambda i: (0, 0)),
        ],
        out_specs=[
            pl.BlockSpec((ROW_BLK, OUT), lambda i: (i, 0)),
            pl.BlockSpec((ROW_BLK, OUT), lambda i: (i, 0)),
        ],
        out_shape=[
            jax.ShapeDtypeStruct((NA, OUT), jnp.float32),
            jax.ShapeDtypeStruct((NA, OUT), jnp.float32),
        ],
    )(hB, hC, W_out, b_out.reshape(1, OUT))


def _tail_reduce_kernel(acc_ref, ha_ref, has_ref, wl_ref, bl_ref,
                        out_ref, fa1_ref, fa2_ref):
    i = pl.program_id(0)
    ha = ha_ref[...]
    fas = []
    for mp in range(2):
        h0 = acc_ref[mp, 0, 0] + acc_ref[mp, 0, 1]
        h1 = acc_ref[mp, 1, 0] + acc_ref[mp, 1, 1]
        h = jnp.concatenate([h0, h1], axis=-1)
        h = (h + ha * has_ref[:, mp:mp + 1]) * (1.0 / 3.0)
        fas.append(jnp.where(h > 0.0, h, jnp.exp(h) - 1.0))
    fa1_ref[...] = fas[0]
    fa2_ref[...] = fas[1]
    t1 = jnp.tanh(jnp.dot(fas[0], wl_ref[...],
                          preferred_element_type=jnp.float32) + bl_ref[...])
    t2 = jnp.tanh(jnp.dot(fas[1], wl_ref[...],
                          preferred_element_type=jnp.float32) + bl_ref[...])
    part = jnp.stack([jnp.sum(t1, axis=0), jnp.sum(t2, axis=0)], axis=0)

    @pl.when(i == 0)
    def _init():
        out_ref[...] = jnp.zeros_like(out_ref)

    out_ref[...] += part


def _tail_out_kernel(beta_ref, fa1_ref, fa2_ref, wo_ref,
                     bo_ref, featA_ref, outA_ref):
    b0 = beta_ref[0]
    b1 = beta_ref[1]
    featA = b0 * fa1_ref[...] + b1 * fa2_ref[...]
    featA_ref[...] = featA
    outA_ref[...] = jnp.dot(featA, wo_ref[...],
                            preferred_element_type=jnp.float32) + bo_ref[...]


def _tail(acc, hA, has, Wl, bl, v, W_out, b_out):
    grid = NA // ROW_BLK
    tsum, fa1, fa2 = pl.pallas_call(
        _tail_reduce_kernel,
        grid=(grid,),
        in_specs=[
            pl.BlockSpec((2, 2, NC, ROW_BLK, DH), lambda i: (0, 0, 0, i, 0)),
            pl.BlockSpec((ROW_BLK, D), lambda i: (i, 0)),
            pl.BlockSpec((ROW_BLK, 2), lambda i: (i, 0)),
            pl.BlockSpec((D, ATTN), lambda i: (0, 0)),
            pl.BlockSpec((1, ATTN), lambda i: (0, 0)),
        ],
        out_specs=[
            pl.BlockSpec((2, ATTN), lambda i: (0, 0)),
            pl.BlockSpec((ROW_BLK, D), lambda i: (i, 0)),
            pl.BlockSpec((ROW_BLK, D), lambda i: (i, 0)),
        ],
        out_shape=[
            jax.ShapeDtypeStruct((2, ATTN), jnp.float32),
            jax.ShapeDtypeStruct((NA, D), jnp.float32),
            jax.ShapeDtypeStruct((NA, D), jnp.float32),
        ],
    )(acc, hA, has, Wl, bl.reshape(1, ATTN))
    scores = (tsum / NA) @ v
    beta = jax.nn.softmax(scores)
    featA, outA = pl.pallas_call(
        _tail_out_kernel,
        grid=(grid,),
        in_specs=[
            pl.BlockSpec(memory_space=pltpu.SMEM),
            pl.BlockSpec((ROW_BLK, D), lambda i: (i, 0)),
            pl.BlockSpec((ROW_BLK, D), lambda i: (i, 0)),
            pl.BlockSpec((D, OUT), lambda i: (0, 0)),
            pl.BlockSpec((1, OUT), lambda i: (0, 0)),
        ],
        out_specs=[
            pl.BlockSpec((ROW_BLK, D), lambda i: (i, 0)),
            pl.BlockSpec((ROW_BLK, OUT), lambda i: (i, 0)),
        ],
        out_shape=[
            jax.ShapeDtypeStruct((NA, D), jnp.float32),
            jax.ShapeDtypeStruct((NA, OUT), jnp.float32),
        ],
    )(beta, fa1, fa2, W_out, b_out.reshape(1, OUT))
    return featA, outA


def _rotate(h, r):
    hc = h.reshape(-1, D // 2, 2)
    hr, hi = hc[:, :, 0], hc[:, :, 1]
    rr, ri = r[:, 0], r[:, 1]
    return jnp.stack([hr * rr - hi * ri, hr * ri + hi * rr], axis=2).reshape(-1, D)


def _pad_cols(idx):
    npad = NI_PAD - NI
    j = jnp.arange(npad, dtype=jnp.int32)
    i0 = jnp.concatenate([idx[:, 0].astype(jnp.int32), NA + (j % NJUNK)])
    i1 = jnp.concatenate([idx[:, 1].astype(jnp.int32), j % NA])
    i2 = jnp.concatenate([idx[:, 2].astype(jnp.int32), j % NA])
    return (i0.reshape(IROWS, 128), i1.reshape(IROWS, 128),
            i2.reshape(IROWS, 128))


def _ptab(p):
    return jnp.concatenate(
        [p.reshape(NA * H), jnp.zeros((PTAB - NA * H,), jnp.float32)])


def kernel(hA, hB, hC, idx_ABA, idx_ACA, attn_r_ABA, attn_r_ACA,
           rAB, rBA, rAC, rCA, Wl, bl, v, W_out, b_out):
    outB, outC = _bc_out(hB, hC, W_out, b_out)
    hBrot = _rotate(hB, rAB)
    hCrot = _rotate(hC, rAC)
    hA4 = hA.reshape(NA, H, F)
    pa1 = jnp.einsum("nhf,hf->nh", hA4, attn_r_ABA[0])
    pb1 = jnp.einsum("nhf,hf->nh", hBrot.reshape(NA, H, F), attn_r_ABA[0])
    pa2 = jnp.einsum("nhf,hf->nh", hA4, attn_r_ACA[0])
    pc2 = jnp.einsum("nhf,hf->nh", hCrot.reshape(NA, H, F), attn_r_ACA[0])

    tab = jnp.stack([jnp.stack([_ptab(pa1), _ptab(pb1)]),
                     jnp.stack([_ptab(pa2), _ptab(pc2)])])
    idxs = jnp.stack([jnp.stack(_pad_cols(idx_ABA)),
                      jnp.stack(_pad_cols(idx_ACA))])

    s_pair, ex = _pass_a(tab, idxs)
    s_tot = s_pair[:, 0] + s_pair[:, 1]
    hAh = jnp.stack([hA[:, :DH], hA[:, DH:]])
    hroth = jnp.stack([hBrot[:, :DH], hBrot[:, DH:],
                       hCrot[:, :DH], hCrot[:, DH:]])
    acc = _pass_b(hAh, hroth, ex, s_tot, idxs)
    has = (s_tot[:, 0] > 0.0).astype(jnp.float32).T

    featA, outA = _tail(acc, hA, has, Wl, bl, v, W_out, b_out)
    return (outA, outB, outC, featA, hB, hC)
